# baseline (speedup 1.0000x reference)
import functools

import jax
import jax.numpy as jnp
from jax import lax
from jax.experimental import pallas as pl
from jax.experimental.pallas import tpu as pltpu
from jax.experimental.pallas import tpu_sc as plsc

N_CORES = 2
N_SUBCORES = 16
LANES = 16
CHUNK = 128
BIG = 20
HALF = 16
GROUP = 4
SLOTS = 2


def _sc_segment_sum(tg2, gen2, bus2, num_bus, agg_rows, r_tile):
    n_stages = r_tile // BIG
    assert n_stages % 2 == 0 and BIG % GROUP == 0
    ng = BIG // GROUP
    mesh = plsc.VectorSubcoreMesh(core_axis_name="c", subcore_axis_name="s")

    @functools.partial(
        pl.kernel,
        out_type=jax.ShapeDtypeStruct((N_CORES, agg_rows, HALF), jnp.float32),
        mesh=mesh,
        compiler_params=pltpu.CompilerParams(use_tc_tiling_on_sc=False),
        scratch_types=[
            pltpu.VMEM_SHARED((agg_rows, HALF), jnp.float32),
            pltpu.VMEM((BIG * CHUNK,), jnp.int32),
            pltpu.VMEM((BIG, CHUNK), jnp.int32),
            pltpu.VMEM((BIG * CHUNK,), jnp.int32),
            pltpu.VMEM((BIG, CHUNK), jnp.int32),
            pltpu.VMEM((SLOTS, GROUP * CHUNK, HALF), jnp.float32),
            pltpu.SemaphoreType.DMA,
            pltpu.SemaphoreType.DMA,
            pltpu.SemaphoreType.DMA,
            pltpu.SemaphoreType.DMA,
            pltpu.SemaphoreType.DMA,
            pltpu.SemaphoreType.DMA,
            pltpu.SemaphoreType.DMA,
            pltpu.SemaphoreType.DMA,
            pltpu.SemaphoreType.DMA,
        ],
    )
    def sc_kernel(tg2_hbm, gen2_hbm, bus2_hbm, out_hbm,
                  agg_sh, gen_a, bus_a, gen_b, bus_b, rows,
                  gs0, gs1, gs2, gs3, ss0, ss1, ss2, ss3, xsem):
        c = lax.axis_index("c")
        s = lax.axis_index("s")
        gs = [gs0, gs1, gs2, gs3]
        ss = [ss0, ss1, ss2, ss3]

        @pl.loop(0, GROUP * CHUNK)
        def _(i):
            rows[0, i, :] = jnp.zeros((LANES,), jnp.float32)

        n_zero = agg_rows // (GROUP * CHUNK)

        @pl.loop(s, n_zero, step=N_SUBCORES)
        def _(k):
            pltpu.async_copy(
                rows.at[0], agg_sh.at[pl.ds(k * GROUP * CHUNK, GROUP * CHUNK)],
                xsem)

        @pl.loop(s, n_zero, step=N_SUBCORES)
        def _(k):
            pltpu.make_async_copy(
                rows.at[0],
                agg_sh.at[pl.ds(k * GROUP * CHUNK, GROUP * CHUNK)],
                xsem).wait()

        plsc.subcore_barrier()

        cvec = jnp.zeros((LANES,), jnp.int32) + c

        def xform(gen_st, p):
            @pl.loop(0, GROUP * CHUNK // LANES)
            def _(q):
                o = p * GROUP * CHUNK + q * LANES
                v = gen_st[pl.ds(o, LANES)]
                gen_st[pl.ds(o, LANES)] = v + v + cvec

        def g_fire(gen_st, p, slot):
            pltpu.async_copy(
                tg2_hbm.at[gen_st.at[pl.ds(p * GROUP * CHUNK, GROUP * CHUNK)]],
                rows.at[slot], gs[slot])

        def g_drain(gen_st, p, slot):
            pltpu.make_async_copy(
                tg2_hbm.at[gen_st.at[pl.ds(p * GROUP * CHUNK, GROUP * CHUNK)]],
                rows.at[slot], gs[slot]).wait()

        def s_fire(bus_st, p, slot):
            for k in range(GROUP):
                pltpu.async_copy(rows.at[slot, pl.ds(k * CHUNK, CHUNK)],
                                 agg_sh.at[bus_st.at[p * GROUP + k]],
                                 ss[slot], add=True)

        def s_drain(bus_st, p, slot):
            for k in range(GROUP):
                pltpu.make_async_copy(rows.at[slot, pl.ds(k * CHUNK, CHUNK)],
                                      agg_sh.at[bus_st.at[p * GROUP + k]],
                                      ss[slot]).wait()

        def do_stage(gen_st, bus_st):
            xform(gen_st, 0)
            g_fire(gen_st, 0, 0)
            for p in range(ng):
                slot = p % SLOTS
                if p + 1 < ng:
                    nslot = (p + 1) % SLOTS
                    xform(gen_st, p + 1)
                    if p >= 1:
                        s_drain(bus_st, p - 1, nslot)
                    g_fire(gen_st, p + 1, nslot)
                g_drain(gen_st, p, slot)
                s_fire(bus_st, p, slot)
            for p in range(max(0, ng - SLOTS), ng):
                s_drain(bus_st, p, p % SLOTS)

        def i_fire(stage, gen_st, bus_st):
            srow = s * r_tile + stage * BIG
            pltpu.async_copy(
                gen2_hbm.at[pl.ds(srow * CHUNK, BIG * CHUNK)], gen_st, xsem)
            pltpu.async_copy(bus2_hbm.at[pl.ds(srow, BIG)], bus_st, xsem)

        def i_drain(stage, gen_st, bus_st):
            srow = s * r_tile + stage * BIG
            pltpu.make_async_copy(
                gen2_hbm.at[pl.ds(srow * CHUNK, BIG * CHUNK)],
                gen_st, xsem).wait()
            pltpu.make_async_copy(
                bus2_hbm.at[pl.ds(srow, BIG)], bus_st, xsem).wait()

        i_fire(0, gen_a, bus_a)

        @pl.loop(0, n_stages // 2)
        def _(u):
            st2 = 2 * u
            i_drain(st2, gen_a, bus_a)
            i_fire(st2 + 1, gen_b, bus_b)
            do_stage(gen_a, bus_a)
            i_drain(st2 + 1, gen_b, bus_b)

            @pl.when(st2 + 2 < n_stages)
            def _():
                i_fire(st2 + 2, gen_a, bus_a)

            do_stage(gen_b, bus_b)

        plsc.subcore_barrier()

        rpt = agg_rows // N_SUBCORES
        pltpu.sync_copy(agg_sh.at[pl.ds(s * rpt, rpt)],
                        out_hbm.at[c, pl.ds(s * rpt, rpt)])

    return sc_kernel(tg2, gen2, bus2)


def _bus_mse(pbf, tbf):
    rows = pbf.shape[0]
    bm = 1000
    grid = rows // bm

    def body(p_ref, t_ref, out_ref, acc_ref):
        i = pl.program_id(0)

        @pl.when(i == 0)
        def _():
            acc_ref[0] = 0.0

        d = p_ref[...] - t_ref[...]
        acc_ref[0] += jnp.sum(d * d)

        @pl.when(i == grid - 1)
        def _():
            out_ref[0, 0] = acc_ref[0]

    return pl.pallas_call(
        body,
        grid=(grid,),
        in_specs=[
            pl.BlockSpec((bm, 128), lambda i: (i, 0)),
            pl.BlockSpec((bm, 128), lambda i: (i, 0)),
        ],
        out_specs=pl.BlockSpec(memory_space=pltpu.SMEM),
        out_shape=jax.ShapeDtypeStruct((1, 1), jnp.float32),
        scratch_shapes=[pltpu.SMEM((1,), jnp.float32)],
    )(pbf, tbf)


def _gen_mse_combine(aggf, pgf, mgf, bus_part, d_tot):
    rows = pgf.shape[1]
    bm = 1568
    grid = rows // bm

    def body(a_ref, p_ref, m_ref, b_ref, out_ref, acc_ref):
        i = pl.program_id(0)

        @pl.when(i == 0)
        def _():
            acc_ref[0] = 0.0
            acc_ref[1] = 0.0

        m = m_ref[...]
        d0 = p_ref[0] - a_ref[0]
        d1 = p_ref[1] - a_ref[1]
        acc_ref[0] += jnp.sum((d0 * d0 + d1 * d1) * m)
        acc_ref[1] += jnp.sum(m)

        @pl.when(i == grid - 1)
        def _():
            cnt = acc_ref[1] * (1.0 / 16.0)
            out_ref[0, 0] = (acc_ref[0] + b_ref[0, 0]) / (cnt * d_tot)

    return pl.pallas_call(
        body,
        grid=(grid,),
        in_specs=[
            pl.BlockSpec((N_CORES, bm, 128), lambda i: (0, i, 0)),
            pl.BlockSpec((N_CORES, bm, 128), lambda i: (0, i, 0)),
            pl.BlockSpec((bm, 128), lambda i: (i, 0)),
            pl.BlockSpec(memory_space=pltpu.SMEM),
        ],
        out_specs=pl.BlockSpec(memory_space=pltpu.SMEM),
        out_shape=jax.ShapeDtypeStruct((1, 1), jnp.float32),
        scratch_shapes=[pltpu.SMEM((2,), jnp.float32)],
    )(aggf, pgf, mgf, bus_part)


def kernel(pred, target_bus, target_gen, edge_index, mask):
    num_bus, d_bus = target_bus.shape
    num_gen, d_gen = target_gen.shape
    n_edges = edge_index.shape[1]

    gen_idx = edge_index[0].astype(jnp.int32)
    bus_idx = edge_index[1].astype(jnp.int32)

    tile_edges = 2 * BIG * CHUNK
    r_tile = -(-n_edges // (N_SUBCORES * tile_edges)) * 2 * BIG
    r_tot = N_SUBCORES * r_tile
    ep = r_tot * CHUNK
    pad = ep - n_edges
    genp = jnp.concatenate([gen_idx, jnp.zeros((pad,), jnp.int32)])
    busp = jnp.concatenate([bus_idx, jnp.full((pad,), num_bus, jnp.int32)])
    gen2 = genp
    bus2 = busp.reshape(r_tot, CHUNK)
    tg2 = target_gen.reshape(num_gen * 2, HALF)

    agg_rows = -(-(num_bus + 1) // (GROUP * CHUNK)) * (GROUP * CHUNK)

    agg = _sc_segment_sum(tg2, gen2, bus2, num_bus, agg_rows, r_tile)

    m = mask.astype(jnp.float32)[:, None]
    pbf = (pred[:, :d_bus] * m).reshape(num_bus * d_bus // 128, 128)
    tbf = (target_bus * m).reshape(num_bus * d_bus // 128, 128)
    grows = num_bus * HALF // 128
    frows = agg_rows * HALF // 128
    gpad = frows - grows
    pgf = jnp.pad(jnp.stack([
        pred[:, d_bus:d_bus + HALF].reshape(grows, 128),
        pred[:, d_bus + HALF:].reshape(grows, 128)]),
        ((0, 0), (0, gpad), (0, 0)))
    mgf = jnp.pad(jnp.broadcast_to(m, (num_bus, HALF)).reshape(grows, 128),
                  ((0, gpad), (0, 0)))
    aggf = agg.reshape(N_CORES, frows, 128)

    bus_part = _bus_mse(pbf, tbf)
    out = _gen_mse_combine(aggf, pgf, mgf, bus_part, pred.shape[1])
    return out[0, 0]


# speedup vs baseline: 1.0189x; 1.0189x over previous
import functools

import jax
import jax.numpy as jnp
from jax import lax
from jax.experimental import pallas as pl
from jax.experimental.pallas import tpu as pltpu
from jax.experimental.pallas import tpu_sc as plsc

N_CORES = 2
N_SUBCORES = 16
LANES = 16
CHUNK = 128
BIG = 20
HALF = 16
GROUP = 4
SLOTS = 2


def _sc_segment_sum(tg2, ei, num_bus, agg_rows, r_tile):
    n_stages = r_tile // BIG
    assert n_stages % 2 == 0 and BIG % GROUP == 0
    ng = BIG // GROUP
    mesh = plsc.VectorSubcoreMesh(core_axis_name="c", subcore_axis_name="s")

    @functools.partial(
        pl.kernel,
        out_type=jax.ShapeDtypeStruct((N_CORES, agg_rows, HALF), jnp.float32),
        mesh=mesh,
        compiler_params=pltpu.CompilerParams(use_tc_tiling_on_sc=False),
        scratch_types=[
            pltpu.VMEM_SHARED((agg_rows, HALF), jnp.float32),
            pltpu.VMEM((BIG, CHUNK), jnp.int32),
            pltpu.VMEM((BIG, CHUNK), jnp.int32),
            pltpu.VMEM((BIG, CHUNK), jnp.int32),
            pltpu.VMEM((BIG, CHUNK), jnp.int32),
            pltpu.VMEM((SLOTS, GROUP * CHUNK, HALF), jnp.float32),
            pltpu.SemaphoreType.DMA,
            pltpu.SemaphoreType.DMA,
            pltpu.SemaphoreType.DMA,
            pltpu.SemaphoreType.DMA,
            pltpu.SemaphoreType.DMA,
            pltpu.SemaphoreType.DMA,
            pltpu.SemaphoreType.DMA,
            pltpu.SemaphoreType.DMA,
            pltpu.SemaphoreType.DMA,
        ],
    )
    def sc_kernel(tg2_hbm, ei_hbm, out_hbm,
                  agg_sh, gen_a, bus_a, gen_b, bus_b, rows,
                  gs0, gs1, gs2, gs3, ss0, ss1, ss2, ss3, xsem):
        c = lax.axis_index("c")
        s = lax.axis_index("s")
        gs = [gs0, gs1, gs2, gs3]
        ss = [ss0, ss1, ss2, ss3]

        @pl.loop(0, GROUP * CHUNK)
        def _(i):
            rows[0, i, :] = jnp.zeros((LANES,), jnp.float32)

        n_zero = agg_rows // (GROUP * CHUNK)

        @pl.loop(s, n_zero, step=N_SUBCORES)
        def _(k):
            pltpu.async_copy(
                rows.at[0], agg_sh.at[pl.ds(k * GROUP * CHUNK, GROUP * CHUNK)],
                xsem)

        @pl.loop(s, n_zero, step=N_SUBCORES)
        def _(k):
            pltpu.make_async_copy(
                rows.at[0],
                agg_sh.at[pl.ds(k * GROUP * CHUNK, GROUP * CHUNK)],
                xsem).wait()

        plsc.subcore_barrier()

        cvec = jnp.zeros((LANES,), jnp.int32) + c

        def xform(gen_st, p):
            for k in range(GROUP):
                j = p * GROUP + k

                @pl.loop(0, CHUNK // LANES)
                def _(q, j=j):
                    v = gen_st[j, pl.ds(q * LANES, LANES)]
                    gen_st[j, pl.ds(q * LANES, LANES)] = v + v + cvec

        def g_fire(gen_st, p, slot):
            for k in range(GROUP):
                pltpu.async_copy(tg2_hbm.at[gen_st.at[p * GROUP + k]],
                                 rows.at[slot, pl.ds(k * CHUNK, CHUNK)],
                                 gs[slot])

        def g_drain(gen_st, p, slot):
            for k in range(GROUP):
                pltpu.make_async_copy(tg2_hbm.at[gen_st.at[p * GROUP + k]],
                                      rows.at[slot, pl.ds(k * CHUNK, CHUNK)],
                                      gs[slot]).wait()

        def s_fire(bus_st, p, slot):
            for k in range(GROUP):
                pltpu.async_copy(rows.at[slot, pl.ds(k * CHUNK, CHUNK)],
                                 agg_sh.at[bus_st.at[p * GROUP + k]],
                                 ss[slot], add=True)

        def s_drain(bus_st, p, slot):
            for k in range(GROUP):
                pltpu.make_async_copy(rows.at[slot, pl.ds(k * CHUNK, CHUNK)],
                                      agg_sh.at[bus_st.at[p * GROUP + k]],
                                      ss[slot]).wait()

        def do_stage(gen_st, bus_st):
            xform(gen_st, 0)
            g_fire(gen_st, 0, 0)
            for p in range(ng):
                slot = p % SLOTS
                if p + 1 < ng:
                    nslot = (p + 1) % SLOTS
                    xform(gen_st, p + 1)
                    if p >= 1:
                        s_drain(bus_st, p - 1, nslot)
                    g_fire(gen_st, p + 1, nslot)
                g_drain(gen_st, p, slot)
                s_fire(bus_st, p, slot)
            for p in range(max(0, ng - SLOTS), ng):
                s_drain(bus_st, p, p % SLOTS)

        def i_fire(stage, gen_st, bus_st):
            srow = s * r_tile + stage * BIG
            pltpu.async_copy(ei_hbm.at[0, pl.ds(srow, BIG)], gen_st, xsem)
            pltpu.async_copy(ei_hbm.at[1, pl.ds(srow, BIG)], bus_st, xsem)

        def i_drain(stage, gen_st, bus_st):
            srow = s * r_tile + stage * BIG
            pltpu.make_async_copy(
                ei_hbm.at[0, pl.ds(srow, BIG)], gen_st, xsem).wait()
            pltpu.make_async_copy(
                ei_hbm.at[1, pl.ds(srow, BIG)], bus_st, xsem).wait()

        i_fire(0, gen_a, bus_a)

        @pl.loop(0, n_stages // 2)
        def _(u):
            st2 = 2 * u
            i_drain(st2, gen_a, bus_a)
            i_fire(st2 + 1, gen_b, bus_b)
            do_stage(gen_a, bus_a)
            i_drain(st2 + 1, gen_b, bus_b)

            @pl.when(st2 + 2 < n_stages)
            def _():
                i_fire(st2 + 2, gen_a, bus_a)

            do_stage(gen_b, bus_b)

        plsc.subcore_barrier()

        rpt = agg_rows // N_SUBCORES
        pltpu.sync_copy(agg_sh.at[pl.ds(s * rpt, rpt)],
                        out_hbm.at[c, pl.ds(s * rpt, rpt)])

    return sc_kernel(tg2, ei)


def _bus_mse(pbf, tbf):
    rows = pbf.shape[0]
    bm = 1000
    grid = rows // bm

    def body(p_ref, t_ref, out_ref, acc_ref):
        i = pl.program_id(0)

        @pl.when(i == 0)
        def _():
            acc_ref[0] = 0.0

        d = p_ref[...] - t_ref[...]
        acc_ref[0] += jnp.sum(d * d)

        @pl.when(i == grid - 1)
        def _():
            out_ref[0, 0] = acc_ref[0]

    return pl.pallas_call(
        body,
        grid=(grid,),
        in_specs=[
            pl.BlockSpec((bm, 128), lambda i: (i, 0)),
            pl.BlockSpec((bm, 128), lambda i: (i, 0)),
        ],
        out_specs=pl.BlockSpec(memory_space=pltpu.SMEM),
        out_shape=jax.ShapeDtypeStruct((1, 1), jnp.float32),
        scratch_shapes=[pltpu.SMEM((1,), jnp.float32)],
    )(pbf, tbf)


def _gen_mse_combine(aggf, pgf, mgf, bus_part, d_tot):
    rows = pgf.shape[1]
    bm = 1568
    grid = rows // bm

    def body(a_ref, p_ref, m_ref, b_ref, out_ref, acc_ref):
        i = pl.program_id(0)

        @pl.when(i == 0)
        def _():
            acc_ref[0] = 0.0
            acc_ref[1] = 0.0

        m = m_ref[...]
        d0 = p_ref[0] - a_ref[0]
        d1 = p_ref[1] - a_ref[1]
        acc_ref[0] += jnp.sum((d0 * d0 + d1 * d1) * m)
        acc_ref[1] += jnp.sum(m)

        @pl.when(i == grid - 1)
        def _():
            cnt = acc_ref[1] * (1.0 / 16.0)
            out_ref[0, 0] = (acc_ref[0] + b_ref[0, 0]) / (cnt * d_tot)

    return pl.pallas_call(
        body,
        grid=(grid,),
        in_specs=[
            pl.BlockSpec((N_CORES, bm, 128), lambda i: (0, i, 0)),
            pl.BlockSpec((N_CORES, bm, 128), lambda i: (0, i, 0)),
            pl.BlockSpec((bm, 128), lambda i: (i, 0)),
            pl.BlockSpec(memory_space=pltpu.SMEM),
        ],
        out_specs=pl.BlockSpec(memory_space=pltpu.SMEM),
        out_shape=jax.ShapeDtypeStruct((1, 1), jnp.float32),
        scratch_shapes=[pltpu.SMEM((2,), jnp.float32)],
    )(aggf, pgf, mgf, bus_part)


def kernel(pred, target_bus, target_gen, edge_index, mask):
    num_bus, d_bus = target_bus.shape
    num_gen, d_gen = target_gen.shape
    n_edges = edge_index.shape[1]

    tile_edges = 2 * BIG * CHUNK
    r_tile = -(-n_edges // (N_SUBCORES * tile_edges)) * 2 * BIG
    r_tot = N_SUBCORES * r_tile
    pad = r_tot * CHUNK - n_edges
    ei = jnp.pad(edge_index.astype(jnp.int32), ((0, 0), (0, pad)),
                 constant_values=num_gen).reshape(2, r_tot, CHUNK)
    tg2 = jnp.pad(target_gen, ((0, 1), (0, 0))).reshape(num_gen * 2 + 2, HALF)

    agg_rows = (-(-max(num_bus, num_gen + 1) // (GROUP * CHUNK))
                * (GROUP * CHUNK))

    agg = _sc_segment_sum(tg2, ei, num_bus, agg_rows, r_tile)

    m = mask.astype(jnp.float32)[:, None]
    pbf = (pred[:, :d_bus] * m).reshape(num_bus * d_bus // 128, 128)
    tbf = (target_bus * m).reshape(num_bus * d_bus // 128, 128)
    grows = num_bus * HALF // 128
    frows = agg_rows * HALF // 128
    gpad = frows - grows
    pgf = jnp.pad(jnp.stack([
        pred[:, d_bus:d_bus + HALF].reshape(grows, 128),
        pred[:, d_bus + HALF:].reshape(grows, 128)]),
        ((0, 0), (0, gpad), (0, 0)))
    mgf = jnp.pad(jnp.broadcast_to(m, (num_bus, HALF)).reshape(grows, 128),
                  ((0, gpad), (0, 0)))
    aggf = agg.reshape(N_CORES, frows, 128)

    bus_part = _bus_mse(pbf, tbf)
    out = _gen_mse_combine(aggf, pgf, mgf, bus_part, pred.shape[1])
    return out[0, 0]
